# Initial kernel scaffold; baseline (speedup 1.0000x reference)
import jax
import jax.numpy as jnp
from jax.experimental import pallas as pl


def kernel(boxes, scores, idxs):
    raise NotImplementedError("write your pallas kernel here")


# speedup vs baseline: 12.8461x; 12.8461x over previous
import functools

import jax
import jax.numpy as jnp
from jax import lax
from jax.experimental import pallas as pl
from jax.experimental.pallas import tpu as pltpu

_N = 20000
_PRE = 2000
_POST = 1000
_THR = 0.7
_MINSZ = 0.001
_IMG_W = 800.0
_IMG_H = 800.0

_K = 2048
_B = 128
_NB = _K // _B


def _nms_body(x1c, y1c, x2c, y2c, x1r, y1r, x2r, y2r, keep_ref, iou_s):
    keep_ref[...] = jnp.ones((_NB, _B), jnp.float32)
    lane = lax.broadcasted_iota(jnp.int32, (1, _B), 1)

    def outer(i, _):
        ax1 = x1c[pl.ds(i * _B, _B), :]
        ay1 = y1c[pl.ds(i * _B, _B), :]
        ax2 = x2c[pl.ds(i * _B, _B), :]
        ay2 = y2c[pl.ds(i * _B, _B), :]
        area_a = (ax2 - ax1) * (ay2 - ay1)

        def iou_vs(j):
            bx1 = x1r[pl.ds(j, 1), :]
            by1 = y1r[pl.ds(j, 1), :]
            bx2 = x2r[pl.ds(j, 1), :]
            by2 = y2r[pl.ds(j, 1), :]
            area_b = (bx2 - bx1) * (by2 - by1)
            wx = jnp.maximum(jnp.minimum(ax2, bx2) - jnp.maximum(ax1, bx1), 0.0)
            wy = jnp.maximum(jnp.minimum(ay2, by2) - jnp.maximum(ay1, by1), 0.0)
            inter = wx * wy
            return inter / ((area_a + area_b) - inter + 1e-9)

        iou_s[...] = iou_vs(i)

        def inner(r, kv):
            row = iou_s[pl.ds(r, 1), :]
            kr = jnp.sum(kv * (lane == r).astype(jnp.float32))
            sup = (row > _THR) & (lane > r) & (kr > 0.5)
            return kv * (1.0 - sup.astype(jnp.float32))

        kv = lax.fori_loop(0, _B, inner, keep_ref[pl.ds(i, 1), :])
        keep_ref[pl.ds(i, 1), :] = kv

        def cross(j, _c):
            ind = (iou_vs(j) > _THR).astype(jnp.float32)
            s = jnp.dot(kv, ind, preferred_element_type=jnp.float32)
            rowj = keep_ref[pl.ds(j, 1), :]
            keep_ref[pl.ds(j, 1), :] = rowj * (1.0 - (s > 0.0).astype(jnp.float32))
            return 0

        lax.fori_loop(i + 1, _NB, cross, 0)
        return 0

    lax.fori_loop(0, _NB, outer, 0)


_nms_call = pl.pallas_call(
    _nms_body,
    out_shape=jax.ShapeDtypeStruct((_NB, _B), jnp.float32),
    scratch_shapes=[pltpu.VMEM((_B, _B), jnp.float32)],
)


@jax.jit
def kernel(boxes, scores, idxs):
    top_scores, top_idx = lax.top_k(scores, _PRE)
    b = boxes[top_idx]
    lv = idxs[top_idx]

    bx = jnp.clip(b[:, 0::2], 0.0, _IMG_W)
    by = jnp.clip(b[:, 1::2], 0.0, _IMG_H)
    b = jnp.stack([bx[:, 0], by[:, 0], bx[:, 1], by[:, 1]], axis=1)

    ws = b[:, 2] - b[:, 0]
    hs = b[:, 3] - b[:, 1]
    valid = (ws >= _MINSZ) & (hs >= _MINSZ)
    sc = jnp.where(valid, top_scores, -jnp.inf)

    max_coordinate = b.max()
    offsets = lv.astype(b.dtype) * (max_coordinate + 1.0)
    bn = b + offsets[:, None]
    bn = jnp.pad(bn, ((0, _K - _PRE), (0, 0)))
    cols = [bn[:, c].reshape(_K, 1) for c in range(4)]
    rows = [bn[:, c].reshape(_NB, _B) for c in range(4)]
    keep_f = _nms_call(*cols, *rows)
    keep = (keep_f.reshape(_K)[:_PRE] > 0.5) & valid

    sc_kept = jnp.where(keep, sc, -jnp.inf)
    final_scores, final_idx = lax.top_k(sc_kept, _POST)
    final_boxes = b[final_idx]
    return jnp.concatenate([final_boxes, final_scores[:, None]], axis=1)


# speedup vs baseline: 13.3173x; 1.0367x over previous
import functools

import jax
import jax.numpy as jnp
from jax import lax
from jax.experimental import pallas as pl
from jax.experimental.pallas import tpu as pltpu

_N = 20000
_PRE = 2000
_POST = 1000
_THR = 0.7
_MINSZ = 0.001
_IMG_W = 800.0
_IMG_H = 800.0

_K = 2048
_B = 128
_NB = _K // _B


def _nms_body(x1c, y1c, x2c, y2c, x1r, y1r, x2r, y2r, keep_ref):
    keep_ref[...] = jnp.ones((_NB, _B), jnp.float32)
    lane = lax.broadcasted_iota(jnp.int32, (1, _B), 1)
    ut = (lax.broadcasted_iota(jnp.int32, (_B, _B), 1)
          > lax.broadcasted_iota(jnp.int32, (_B, _B), 0)).astype(jnp.float32)

    def outer(i, _):
        ax1 = x1c[pl.ds(i * _B, _B), :]
        ay1 = y1c[pl.ds(i * _B, _B), :]
        ax2 = x2c[pl.ds(i * _B, _B), :]
        ay2 = y2c[pl.ds(i * _B, _B), :]
        area_a = (ax2 - ax1) * (ay2 - ay1)

        def iou_vs(j):
            bx1 = x1r[pl.ds(j, 1), :]
            by1 = y1r[pl.ds(j, 1), :]
            bx2 = x2r[pl.ds(j, 1), :]
            by2 = y2r[pl.ds(j, 1), :]
            area_b = (bx2 - bx1) * (by2 - by1)
            wx = jnp.maximum(jnp.minimum(ax2, bx2) - jnp.maximum(ax1, bx1), 0.0)
            wy = jnp.maximum(jnp.minimum(ay2, by2) - jnp.maximum(ay1, by1), 0.0)
            inter = wx * wy
            return inter / ((area_a + area_b) - inter + 1e-9)

        supm = (iou_vs(i) > _THR).astype(jnp.float32) * ut
        kv = keep_ref[pl.ds(i, 1), :]
        for r in range(_B):
            kr = jnp.sum(jnp.where(lane == r, kv, 0.0))
            row = lax.slice(supm, (r, 0), (r + 1, _B))
            kv = kv * (1.0 - row * kr)
        keep_ref[pl.ds(i, 1), :] = kv

        def cross(j, _c):
            ind = (iou_vs(j) > _THR).astype(jnp.float32)
            s = jnp.dot(kv, ind, preferred_element_type=jnp.float32)
            rowj = keep_ref[pl.ds(j, 1), :]
            keep_ref[pl.ds(j, 1), :] = rowj * (1.0 - (s > 0.0).astype(jnp.float32))
            return 0

        lax.fori_loop(i + 1, _NB, cross, 0)
        return 0

    lax.fori_loop(0, _NB, outer, 0)


_nms_call = pl.pallas_call(
    _nms_body,
    out_shape=jax.ShapeDtypeStruct((_NB, _B), jnp.float32),
)


@jax.jit
def kernel(boxes, scores, idxs):
    top_scores, top_idx = lax.top_k(scores, _PRE)
    b = boxes[top_idx]
    lv = idxs[top_idx]

    bx = jnp.clip(b[:, 0::2], 0.0, _IMG_W)
    by = jnp.clip(b[:, 1::2], 0.0, _IMG_H)
    b = jnp.stack([bx[:, 0], by[:, 0], bx[:, 1], by[:, 1]], axis=1)

    ws = b[:, 2] - b[:, 0]
    hs = b[:, 3] - b[:, 1]
    valid = (ws >= _MINSZ) & (hs >= _MINSZ)
    sc = jnp.where(valid, top_scores, -jnp.inf)

    max_coordinate = b.max()
    offsets = lv.astype(b.dtype) * (max_coordinate + 1.0)
    bn = b + offsets[:, None]
    bn = jnp.pad(bn, ((0, _K - _PRE), (0, 0)))
    cols = [bn[:, c].reshape(_K, 1) for c in range(4)]
    rows = [bn[:, c].reshape(_NB, _B) for c in range(4)]
    keep_f = _nms_call(*cols, *rows)
    keep = (keep_f.reshape(_K)[:_PRE] > 0.5) & valid

    sc_kept = jnp.where(keep, sc, -jnp.inf)
    final_scores, final_idx = lax.top_k(sc_kept, _POST)
    final_boxes = b[final_idx]
    return jnp.concatenate([final_boxes, final_scores[:, None]], axis=1)


# speedup vs baseline: 43.4449x; 3.2623x over previous
import functools

import jax
import jax.numpy as jnp
from jax import lax
from jax.experimental import pallas as pl
from jax.experimental.pallas import tpu as pltpu

_N = 20000
_PRE = 2000
_POST = 1000
_THR = 0.7
_MINSZ = 0.001
_IMG_W = 800.0
_IMG_H = 800.0

_K = 2048
_B = 128
_NB = _K // _B


def _nms_body(x1c, y1c, x2c, y2c, x1r, y1r, x2r, y2r, keep_ref):
    keep_ref[...] = jnp.ones((_NB, _B), jnp.float32)
    lane = lax.broadcasted_iota(jnp.int32, (1, _B), 1)
    ut = (lax.broadcasted_iota(jnp.int32, (_B, _B), 1)
          > lax.broadcasted_iota(jnp.int32, (_B, _B), 0)).astype(jnp.float32)

    def outer(i, _):
        ax1 = x1c[pl.ds(i * _B, _B), :]
        ay1 = y1c[pl.ds(i * _B, _B), :]
        ax2 = x2c[pl.ds(i * _B, _B), :]
        ay2 = y2c[pl.ds(i * _B, _B), :]
        area_a = (ax2 - ax1) * (ay2 - ay1)

        def iou_vs(j):
            bx1 = x1r[pl.ds(j, 1), :]
            by1 = y1r[pl.ds(j, 1), :]
            bx2 = x2r[pl.ds(j, 1), :]
            by2 = y2r[pl.ds(j, 1), :]
            area_b = (bx2 - bx1) * (by2 - by1)
            wx = jnp.maximum(jnp.minimum(ax2, bx2) - jnp.maximum(ax1, bx1), 0.0)
            wy = jnp.maximum(jnp.minimum(ay2, by2) - jnp.maximum(ay1, by1), 0.0)
            inter = wx * wy
            return inter / ((area_a + area_b) - inter + 1e-9)

        supm = (iou_vs(i) > _THR).astype(jnp.float32) * ut
        init = keep_ref[pl.ds(i, 1), :]

        def fp_cond(st):
            return st[0]

        def fp_body(st):
            _, kv = st
            s = jnp.dot(kv, supm, preferred_element_type=jnp.float32)
            kv2 = jnp.where(s > 0.0, 0.0, init)
            return jnp.any(kv2 != kv), kv2

        kv = lax.while_loop(fp_cond, fp_body, (jnp.bool_(True), init))[1]
        keep_ref[pl.ds(i, 1), :] = kv

        def cross(j, _c):
            ind = (iou_vs(j) > _THR).astype(jnp.float32)
            s = jnp.dot(kv, ind, preferred_element_type=jnp.float32)
            rowj = keep_ref[pl.ds(j, 1), :]
            keep_ref[pl.ds(j, 1), :] = rowj * (1.0 - (s > 0.0).astype(jnp.float32))
            return 0

        lax.fori_loop(i + 1, _NB, cross, 0)
        return 0

    lax.fori_loop(0, _NB, outer, 0)


_nms_call = pl.pallas_call(
    _nms_body,
    out_shape=jax.ShapeDtypeStruct((_NB, _B), jnp.float32),
)


@jax.jit
def kernel(boxes, scores, idxs):
    top_scores, top_idx = lax.top_k(scores, _PRE)
    b = boxes[top_idx]
    lv = idxs[top_idx]

    bx = jnp.clip(b[:, 0::2], 0.0, _IMG_W)
    by = jnp.clip(b[:, 1::2], 0.0, _IMG_H)
    b = jnp.stack([bx[:, 0], by[:, 0], bx[:, 1], by[:, 1]], axis=1)

    ws = b[:, 2] - b[:, 0]
    hs = b[:, 3] - b[:, 1]
    valid = (ws >= _MINSZ) & (hs >= _MINSZ)
    sc = jnp.where(valid, top_scores, -jnp.inf)

    max_coordinate = b.max()
    offsets = lv.astype(b.dtype) * (max_coordinate + 1.0)
    bn = b + offsets[:, None]
    bn = jnp.pad(bn, ((0, _K - _PRE), (0, 0)))
    cols = [bn[:, c].reshape(_K, 1) for c in range(4)]
    rows = [bn[:, c].reshape(_NB, _B) for c in range(4)]
    keep_f = _nms_call(*cols, *rows)
    keep = (keep_f.reshape(_K)[:_PRE] > 0.5) & valid

    sc_kept = jnp.where(keep, sc, -jnp.inf)
    final_scores, final_idx = lax.top_k(sc_kept, _POST)
    final_boxes = b[final_idx]
    return jnp.concatenate([final_boxes, final_scores[:, None]], axis=1)
